# baseline (speedup 1.0000x reference)
import jax
import jax.numpy as jnp
from jax.experimental import pallas as pl
from jax.experimental.pallas import tpu as pltpu

NUM_CLASSES = 1000
ROWS = 4096
COLS = 26
BLOCK_ROWS = 32


def _onehot_block(x_ref, o_ref):
    idx = x_ref[...]
    iota = jax.lax.broadcasted_iota(jnp.int32, (BLOCK_ROWS, COLS, NUM_CLASSES), 2)
    o_ref[...] = (iota == idx[:, :, None]).astype(jnp.float32)


def kernel(x):
    x = x.astype(jnp.int32)
    grid = (ROWS // BLOCK_ROWS,)
    out = pl.pallas_call(
        _onehot_block,
        grid=grid,
        in_specs=[pl.BlockSpec((BLOCK_ROWS, COLS), lambda i: (i, 0))],
        out_specs=pl.BlockSpec((BLOCK_ROWS, COLS, NUM_CLASSES), lambda i: (i, 0, 0)),
        out_shape=jax.ShapeDtypeStruct((ROWS, COLS, NUM_CLASSES), jnp.float32),
    )(x)
    return out
